# baseline (speedup 1.0000x reference)
import functools

import jax
import jax.numpy as jnp
from jax import lax
from jax.experimental import pallas as pl
from jax.experimental.pallas import tpu as pltpu
from jax.experimental.pallas import tpu_sc as plsc

NUM_CORES = 2
NUM_SUBCORES = 16
NUM_WORKERS = NUM_CORES * NUM_SUBCORES


@functools.partial(jax.jit, static_argnums=(3, 4, 5))
def _sc_embed(x, token_table, pos_table, batch, seq_len, d):
  b_per_w = batch // NUM_WORKERS
  mesh = plsc.VectorSubcoreMesh(core_axis_name="c", subcore_axis_name="s")

  @functools.partial(
      pl.kernel,
      mesh=mesh,
      compiler_params=pltpu.CompilerParams(use_tc_tiling_on_sc=False),
      out_type=jax.ShapeDtypeStruct((batch, seq_len, d), jnp.float32),
      scratch_types=[
          pltpu.VMEM((b_per_w, seq_len), jnp.int32),
          pltpu.VMEM((seq_len, d), jnp.float32),
          pltpu.VMEM((2, seq_len, d), jnp.float32),
          pltpu.SemaphoreType.DMA,
          pltpu.SemaphoreType.DMA,
          pltpu.SemaphoreType.DMA,
          pltpu.SemaphoreType.DMA,
      ],
  )
  def k(x_hbm, table_hbm, pos_hbm, out_hbm, idx_v, pos_v, rows_v,
        gsem0, gsem1, osem0, osem1):
    wid = lax.axis_index("s") * NUM_CORES + lax.axis_index("c")
    b0 = wid * b_per_w
    pltpu.sync_copy(x_hbm.at[pl.ds(b0, b_per_w), :], idx_v)
    pltpu.sync_copy(pos_hbm, pos_v)

    gsems = (gsem0, gsem1)
    osems = (osem0, osem1)

    def gather_copies(bb, par, sem):
      return (
          pltpu.make_async_copy(
              table_hbm.at[idx_v.at[bb, pl.ds(0, 128)]],
              rows_v.at[par, pl.ds(0, 128)], sem),
          pltpu.make_async_copy(
              table_hbm.at[idx_v.at[bb, pl.ds(128, seq_len - 128)]],
              rows_v.at[par, pl.ds(128, seq_len - 128)], sem),
      )

    def fire_gather(bb, par, sem):
      for cp in gather_copies(bb, par, sem):
        cp.start()

    fire_gather(0, 0, gsem0)

    def seq_body(bb, par):
      @pl.when(bb + 1 < b_per_w)
      def _():
        @pl.when(bb >= 1)
        def _():
          pltpu.make_async_copy(
              rows_v.at[1 - par], out_hbm.at[b0 + bb - 1],
              osems[1 - par]).wait()
        fire_gather(bb + 1, 1 - par, gsems[1 - par])

      for cp in gather_copies(bb, par, gsems[par]):
        cp.wait()

      def add_rows(i, c):
        r = i * 8
        for uu in range(8):
          for h in range(d // 16):
            sl = pl.ds(h * 16, 16)
            rows_v[par, r + uu, sl] = rows_v[par, r + uu, sl] + pos_v[r + uu, sl]
        return c

      lax.fori_loop(0, seq_len // 8, add_rows, 0)
      pltpu.async_copy(rows_v.at[par], out_hbm.at[b0 + bb], osems[par])

    def pair_body(p, carry):
      seq_body(p * 2, 0)
      seq_body(p * 2 + 1, 1)
      return carry

    lax.fori_loop(0, b_per_w // 2, pair_body, 0)

    for par in range(2):
      bb_last = b_per_w - 2 + par
      pltpu.make_async_copy(
          rows_v.at[par], out_hbm.at[b0 + bb_last], osems[par]).wait()

  return k(x, token_table, pos_table)


def kernel(x, token_table, pos_table):
  batch, seq_len = x.shape
  d = token_table.shape[1]
  return _sc_embed(x.astype(jnp.int32), token_table, pos_table,
                   batch, seq_len, d)


# speedup vs baseline: 1.1359x; 1.1359x over previous
import functools

import jax
import jax.numpy as jnp
from jax import lax
from jax.experimental import pallas as pl
from jax.experimental.pallas import tpu as pltpu
from jax.experimental.pallas import tpu_sc as plsc

NUM_CORES = 2
NUM_SUBCORES = 16
NUM_WORKERS = NUM_CORES * NUM_SUBCORES

TR_CHUNK = 2048


@functools.partial(jax.jit, static_argnums=(1, 2))
def _tc_table_to_rowmajor(table_t, vocab, d):
  grid = (vocab + TR_CHUNK - 1) // TR_CHUNK
  rows_per_chunk = TR_CHUNK * d // 128
  sub = TR_CHUNK // 4

  def body(t_ref, o_ref):
    for kq in range(4):
      o_ref[:, kq * d:(kq + 1) * d] = t_ref[:, kq * sub:(kq + 1) * sub].T

  return pl.pallas_call(
      body,
      grid=(grid,),
      in_specs=[pl.BlockSpec((d, TR_CHUNK), lambda i: (0, i))],
      out_specs=pl.BlockSpec((rows_per_chunk, 128), lambda i: (i, 0)),
      out_shape=jax.ShapeDtypeStruct((grid * rows_per_chunk, 128),
                                     jnp.float32),
  )(table_t)


@functools.partial(jax.jit, static_argnums=(3, 4, 5))
def _sc_embed(x, token_table, pos_table, batch, seq_len, d):
  b_per_w = batch // NUM_WORKERS
  mesh = plsc.VectorSubcoreMesh(core_axis_name="c", subcore_axis_name="s")

  @functools.partial(
      pl.kernel,
      mesh=mesh,
      compiler_params=pltpu.CompilerParams(use_tc_tiling_on_sc=False),
      out_type=jax.ShapeDtypeStruct((batch, seq_len, d), jnp.float32),
      scratch_types=[
          pltpu.VMEM((b_per_w, seq_len), jnp.int32),
          pltpu.VMEM((seq_len, d), jnp.float32),
          pltpu.VMEM((2, seq_len, d), jnp.float32),
          pltpu.SemaphoreType.DMA,
          pltpu.SemaphoreType.DMA,
          pltpu.SemaphoreType.DMA,
          pltpu.SemaphoreType.DMA,
      ],
  )
  def k(x_hbm, table_hbm, pos_hbm, out_hbm, idx_v, pos_v, rows_v,
        gsem0, gsem1, osem0, osem1):
    wid = lax.axis_index("s") * NUM_CORES + lax.axis_index("c")
    b0 = wid * b_per_w
    pltpu.sync_copy(x_hbm.at[pl.ds(b0, b_per_w), :], idx_v)
    pltpu.sync_copy(pos_hbm, pos_v)

    gsems = (gsem0, gsem1)
    osems = (osem0, osem1)

    def gather_copies(bb, par, sem):
      return (
          pltpu.make_async_copy(
              table_hbm.at[idx_v.at[bb, pl.ds(0, 128)]],
              rows_v.at[par, pl.ds(0, 128)], sem),
          pltpu.make_async_copy(
              table_hbm.at[idx_v.at[bb, pl.ds(128, seq_len - 128)]],
              rows_v.at[par, pl.ds(128, seq_len - 128)], sem),
      )

    def fire_gather(bb, par, sem):
      for cp in gather_copies(bb, par, sem):
        cp.start()

    fire_gather(0, 0, gsem0)

    def seq_body(bb, par):
      @pl.when(bb + 1 < b_per_w)
      def _():
        @pl.when(bb >= 1)
        def _():
          pltpu.make_async_copy(
              rows_v.at[1 - par], out_hbm.at[b0 + bb - 1],
              osems[1 - par]).wait()
        fire_gather(bb + 1, 1 - par, gsems[1 - par])

      for cp in gather_copies(bb, par, gsems[par]):
        cp.wait()

      def add_rows(i, c):
        r = i * 8
        for uu in range(8):
          for h in range(d // 16):
            sl = pl.ds(h * 16, 16)
            rows_v[par, r + uu, sl] = rows_v[par, r + uu, sl] + pos_v[r + uu, sl]
        return c

      lax.fori_loop(0, seq_len // 8, add_rows, 0)
      pltpu.async_copy(rows_v.at[par], out_hbm.at[b0 + bb], osems[par])

    def pair_body(p, carry):
      seq_body(p * 2, 0)
      seq_body(p * 2 + 1, 1)
      return carry

    lax.fori_loop(0, b_per_w // 2, pair_body, 0)

    for par in range(2):
      bb_last = b_per_w - 2 + par
      pltpu.make_async_copy(
          rows_v.at[par], out_hbm.at[b0 + bb_last], osems[par]).wait()

  return k(x, token_table, pos_table)


def kernel(x, token_table, pos_table):
  batch, seq_len = x.shape
  vocab, d = token_table.shape
  table_lin = _tc_table_to_rowmajor(token_table.T, vocab, d)
  table_lin = table_lin.reshape((vocab + TR_CHUNK - 1) // TR_CHUNK * TR_CHUNK,
                                d)
  x = x.astype(jnp.int32)
  xm = ((x >> 11) << 11) + ((x & 511) << 2) + ((x >> 9) & 3)
  return _sc_embed(xm, table_lin, pos_table, batch, seq_len, d)


# speedup vs baseline: 1.1841x; 1.0424x over previous
import functools

import jax
import jax.numpy as jnp
from jax import lax
from jax.experimental import pallas as pl
from jax.experimental.pallas import tpu as pltpu
from jax.experimental.pallas import tpu_sc as plsc

NUM_CORES = 2
NUM_SUBCORES = 16
NUM_WORKERS = NUM_CORES * NUM_SUBCORES

TR_CHUNK = 2048


@functools.partial(jax.jit, static_argnums=(2, 3))
def _tc_table_to_rowmajor(table_t, ident, vocab, d):
  grid = (vocab + TR_CHUNK - 1) // TR_CHUNK
  rows_per_chunk = TR_CHUNK * d // 128
  sub = TR_CHUNK // 4

  def body(ident_ref, t_ref, o_ref):
    idm = ident_ref[...]
    for kq in range(4):
      for j in range(sub // 128):
        xq = t_ref[:, pl.ds((kq * (sub // 128) + j) * 128, 128)]
        y = lax.dot_general(idm, xq, (((1,), (1,)), ((), ())),
                            preferred_element_type=jnp.float32)
        o_ref[pl.ds(j * 128, 128), pl.ds(kq * d, d)] = y

  return pl.pallas_call(
      body,
      grid=(grid,),
      in_specs=[pl.BlockSpec((128, 128), lambda i: (0, 0)),
                pl.BlockSpec((d, TR_CHUNK), lambda i: (0, i))],
      out_specs=pl.BlockSpec((rows_per_chunk, 128), lambda i: (i, 0)),
      out_shape=jax.ShapeDtypeStruct((grid * rows_per_chunk, 128),
                                     jnp.float32),
  )(ident, table_t)


@functools.partial(jax.jit, static_argnums=(3, 4, 5))
def _sc_embed(x, token_table, pos_table, batch, seq_len, d):
  b_per_w = batch // NUM_WORKERS
  mesh = plsc.VectorSubcoreMesh(core_axis_name="c", subcore_axis_name="s")

  @functools.partial(
      pl.kernel,
      mesh=mesh,
      compiler_params=pltpu.CompilerParams(use_tc_tiling_on_sc=False),
      out_type=jax.ShapeDtypeStruct((batch, seq_len, d), jnp.float32),
      scratch_types=[
          pltpu.VMEM((b_per_w, seq_len), jnp.int32),
          pltpu.VMEM((seq_len, d), jnp.float32),
          pltpu.VMEM((2, seq_len, d), jnp.float32),
          pltpu.SemaphoreType.DMA,
          pltpu.SemaphoreType.DMA,
          pltpu.SemaphoreType.DMA,
          pltpu.SemaphoreType.DMA,
      ],
  )
  def k(x_hbm, table_hbm, pos_hbm, out_hbm, idx_v, pos_v, rows_v,
        gsem0, gsem1, osem0, osem1):
    wid = lax.axis_index("s") * NUM_CORES + lax.axis_index("c")
    b0 = wid * b_per_w
    pltpu.sync_copy(x_hbm.at[pl.ds(b0, b_per_w), :], idx_v)
    pltpu.sync_copy(pos_hbm, pos_v)

    gsems = (gsem0, gsem1)
    osems = (osem0, osem1)

    def gather_copies(bb, par, sem):
      return (
          pltpu.make_async_copy(
              table_hbm.at[idx_v.at[bb, pl.ds(0, 128)]],
              rows_v.at[par, pl.ds(0, 128)], sem),
          pltpu.make_async_copy(
              table_hbm.at[idx_v.at[bb, pl.ds(128, seq_len - 128)]],
              rows_v.at[par, pl.ds(128, seq_len - 128)], sem),
      )

    def fire_gather(bb, par, sem):
      for cp in gather_copies(bb, par, sem):
        cp.start()

    fire_gather(0, 0, gsem0)

    def seq_body(bb, par):
      @pl.when(bb + 1 < b_per_w)
      def _():
        @pl.when(bb >= 1)
        def _():
          pltpu.make_async_copy(
              rows_v.at[1 - par], out_hbm.at[b0 + bb - 1],
              osems[1 - par]).wait()
        fire_gather(bb + 1, 1 - par, gsems[1 - par])

      for cp in gather_copies(bb, par, gsems[par]):
        cp.wait()

      def add_rows(i, c):
        r = i * 8
        for uu in range(8):
          for h in range(d // 16):
            sl = pl.ds(h * 16, 16)
            rows_v[par, r + uu, sl] = rows_v[par, r + uu, sl] + pos_v[r + uu, sl]
        return c

      lax.fori_loop(0, seq_len // 8, add_rows, 0)
      pltpu.async_copy(rows_v.at[par], out_hbm.at[b0 + bb], osems[par])

    def pair_body(p, carry):
      seq_body(p * 2, 0)
      seq_body(p * 2 + 1, 1)
      return carry

    lax.fori_loop(0, b_per_w // 2, pair_body, 0)

    for par in range(2):
      bb_last = b_per_w - 2 + par
      pltpu.make_async_copy(
          rows_v.at[par], out_hbm.at[b0 + bb_last], osems[par]).wait()

  return k(x, token_table, pos_table)


def kernel(x, token_table, pos_table):
  batch, seq_len = x.shape
  vocab, d = token_table.shape
  ident = jnp.eye(128, dtype=jnp.float32)
  table_lin = _tc_table_to_rowmajor(token_table.T, ident, vocab, d)
  table_lin = table_lin.reshape((vocab + TR_CHUNK - 1) // TR_CHUNK * TR_CHUNK,
                                d)
  x = x.astype(jnp.int32)
  xm = ((x >> 11) << 11) + ((x & 511) << 2) + ((x >> 9) & 3)
  return _sc_embed(xm, table_lin, pos_table, batch, seq_len, d)


# speedup vs baseline: 1.1987x; 1.0124x over previous
import functools

import jax
import jax.numpy as jnp
from jax import lax
from jax.experimental import pallas as pl
from jax.experimental.pallas import tpu as pltpu
from jax.experimental.pallas import tpu_sc as plsc

NUM_CORES = 2
NUM_SUBCORES = 16
NUM_WORKERS = NUM_CORES * NUM_SUBCORES

TR_CHUNK = 2048


@functools.partial(jax.jit, static_argnums=(2, 3))
def _tc_table_to_rowmajor(table_t, ident, vocab, d):
  grid = (vocab + TR_CHUNK - 1) // TR_CHUNK
  rows_per_chunk = TR_CHUNK * d // 128
  sub = TR_CHUNK // 4

  def body(ident_ref, t_ref, o_ref):
    idm = ident_ref[...]
    for j in range(sub // 128):
      ys = []
      for kq in range(4):
        xq = t_ref[:, pl.ds((kq * (sub // 128) + j) * 128, 128)]
        ys.append(lax.dot_general(idm, xq, (((1,), (1,)), ((), ())),
                                  preferred_element_type=jnp.float32))
      o_ref[pl.ds(j * 128, 128), :] = jnp.concatenate(ys, axis=1)

  return pl.pallas_call(
      body,
      grid=(grid,),
      in_specs=[pl.BlockSpec((128, 128), lambda i: (0, 0)),
                pl.BlockSpec((d, TR_CHUNK), lambda i: (0, i))],
      out_specs=pl.BlockSpec((rows_per_chunk, 128), lambda i: (i, 0)),
      out_shape=jax.ShapeDtypeStruct((grid * rows_per_chunk, 128),
                                     jnp.float32),
  )(ident, table_t)


@functools.partial(jax.jit, static_argnums=(3, 4, 5))
def _sc_embed(x, token_table, pos_table, batch, seq_len, d):
  b_per_w = batch // NUM_WORKERS
  mesh = plsc.VectorSubcoreMesh(core_axis_name="c", subcore_axis_name="s")

  @functools.partial(
      pl.kernel,
      mesh=mesh,
      compiler_params=pltpu.CompilerParams(use_tc_tiling_on_sc=False),
      out_type=jax.ShapeDtypeStruct((batch, seq_len, d), jnp.float32),
      scratch_types=[
          pltpu.VMEM((b_per_w, seq_len), jnp.int32),
          pltpu.VMEM((seq_len, d), jnp.float32),
          pltpu.VMEM((2, seq_len, d), jnp.float32),
          pltpu.SemaphoreType.DMA,
          pltpu.SemaphoreType.DMA,
          pltpu.SemaphoreType.DMA,
          pltpu.SemaphoreType.DMA,
      ],
  )
  def k(x_hbm, table_hbm, pos_hbm, out_hbm, idx_v, pos_v, rows_v,
        gsem0, gsem1, osem0, osem1):
    wid = lax.axis_index("s") * NUM_CORES + lax.axis_index("c")
    b0 = wid * b_per_w
    pltpu.sync_copy(x_hbm.at[pl.ds(b0, b_per_w), :], idx_v)
    pltpu.sync_copy(pos_hbm, pos_v)

    gsems = (gsem0, gsem1)
    osems = (osem0, osem1)

    def gather_copies(bb, par, sem):
      return (
          pltpu.make_async_copy(
              table_hbm.at[idx_v.at[bb, pl.ds(0, 128)]],
              rows_v.at[par, pl.ds(0, 128)], sem),
          pltpu.make_async_copy(
              table_hbm.at[idx_v.at[bb, pl.ds(128, seq_len - 128)]],
              rows_v.at[par, pl.ds(128, seq_len - 128)], sem),
      )

    def fire_gather(bb, par, sem):
      for cp in gather_copies(bb, par, sem):
        cp.start()

    fire_gather(0, 0, gsem0)

    def seq_body(bb, par):
      @pl.when(bb + 1 < b_per_w)
      def _():
        @pl.when(bb >= 1)
        def _():
          pltpu.make_async_copy(
              rows_v.at[1 - par], out_hbm.at[b0 + bb - 1],
              osems[1 - par]).wait()
        fire_gather(bb + 1, 1 - par, gsems[1 - par])

      for cp in gather_copies(bb, par, gsems[par]):
        cp.wait()

      def add_rows(i, c):
        r = i * 8
        for uu in range(8):
          for h in range(d // 16):
            sl = pl.ds(h * 16, 16)
            rows_v[par, r + uu, sl] = rows_v[par, r + uu, sl] + pos_v[r + uu, sl]
        return c

      lax.fori_loop(0, seq_len // 8, add_rows, 0)
      pltpu.async_copy(rows_v.at[par], out_hbm.at[b0 + bb], osems[par])

    def pair_body(p, carry):
      seq_body(p * 2, 0)
      seq_body(p * 2 + 1, 1)
      return carry

    lax.fori_loop(0, b_per_w // 2, pair_body, 0)

    for par in range(2):
      bb_last = b_per_w - 2 + par
      pltpu.make_async_copy(
          rows_v.at[par], out_hbm.at[b0 + bb_last], osems[par]).wait()

  return k(x, token_table, pos_table)


def kernel(x, token_table, pos_table):
  batch, seq_len = x.shape
  vocab, d = token_table.shape
  ident = jnp.eye(128, dtype=jnp.float32)
  table_lin = _tc_table_to_rowmajor(token_table.T, ident, vocab, d)
  table_lin = table_lin.reshape((vocab + TR_CHUNK - 1) // TR_CHUNK * TR_CHUNK,
                                d)
  x = x.astype(jnp.int32)
  xm = ((x >> 11) << 11) + ((x & 511) << 2) + ((x >> 9) & 3)
  return _sc_embed(xm, table_lin, pos_table, batch, seq_len, d)


# speedup vs baseline: 1.9107x; 1.5939x over previous
import functools

import jax
import jax.numpy as jnp
from jax import lax
from jax.experimental import pallas as pl
from jax.experimental.pallas import tpu as pltpu
from jax.experimental.pallas import tpu_sc as plsc

NUM_CORES = 2
NUM_SUBCORES = 16
NUM_WORKERS = NUM_CORES * NUM_SUBCORES

TR_CHUNK = 8192


@functools.partial(jax.jit, static_argnums=(2, 3))
def _tc_table_to_rowmajor(table_t, ident, vocab, d):
  grid = (vocab + TR_CHUNK - 1) // TR_CHUNK
  rows_per_chunk = TR_CHUNK * d // 128
  sub = TR_CHUNK // 4

  def body(ident_ref, t_ref, o_ref):
    idm = ident_ref[...]
    for j in range(sub // 128):
      ys = []
      for kq in range(4):
        xq = t_ref[:, pl.ds((kq * (sub // 128) + j) * 128, 128)]
        ys.append(lax.dot_general(idm, xq, (((1,), (1,)), ((), ())),
                                  preferred_element_type=jnp.float32))
      o_ref[pl.ds(j * 128, 128), :] = jnp.concatenate(ys, axis=1)

  return pl.pallas_call(
      body,
      grid=(grid,),
      in_specs=[pl.BlockSpec((128, 128), lambda i: (0, 0)),
                pl.BlockSpec((d, TR_CHUNK), lambda i: (0, i))],
      out_specs=pl.BlockSpec((rows_per_chunk, 128), lambda i: (i, 0)),
      out_shape=jax.ShapeDtypeStruct((grid * rows_per_chunk, 128),
                                     jnp.float32),
  )(ident, table_t)


@functools.partial(jax.jit, static_argnums=(3, 4, 5))
def _sc_embed(x, token_table, pos_table, batch, seq_len, d):
  b_per_w = batch // NUM_WORKERS
  mesh = plsc.VectorSubcoreMesh(core_axis_name="c", subcore_axis_name="s")

  @functools.partial(
      pl.kernel,
      mesh=mesh,
      compiler_params=pltpu.CompilerParams(use_tc_tiling_on_sc=False),
      out_type=jax.ShapeDtypeStruct((batch, seq_len, d), jnp.float32),
      scratch_types=[
          pltpu.VMEM((b_per_w, seq_len), jnp.int32),
          pltpu.VMEM((seq_len, d), jnp.float32),
          pltpu.VMEM((2, seq_len, d), jnp.float32),
          pltpu.SemaphoreType.DMA,
          pltpu.SemaphoreType.DMA,
          pltpu.SemaphoreType.DMA,
          pltpu.SemaphoreType.DMA,
      ],
  )
  def k(x_hbm, table_hbm, pos_hbm, out_hbm, idx_v, pos_v, rows_v,
        gsem0, gsem1, osem0, osem1):
    wid = lax.axis_index("s") * NUM_CORES + lax.axis_index("c")
    b0 = wid * b_per_w
    pltpu.sync_copy(x_hbm.at[pl.ds(b0, b_per_w), :], idx_v)
    pltpu.sync_copy(pos_hbm, pos_v)

    gsems = (gsem0, gsem1)
    osems = (osem0, osem1)

    def gather_copies(bb, par, sem):
      return (
          pltpu.make_async_copy(
              table_hbm.at[idx_v.at[bb, pl.ds(0, 128)]],
              rows_v.at[par, pl.ds(0, 128)], sem),
          pltpu.make_async_copy(
              table_hbm.at[idx_v.at[bb, pl.ds(128, seq_len - 128)]],
              rows_v.at[par, pl.ds(128, seq_len - 128)], sem),
      )

    def fire_gather(bb, par, sem):
      for cp in gather_copies(bb, par, sem):
        cp.start()

    fire_gather(0, 0, gsem0)

    def seq_body(bb, par):
      @pl.when(bb + 1 < b_per_w)
      def _():
        @pl.when(bb >= 1)
        def _():
          pltpu.make_async_copy(
              rows_v.at[1 - par], out_hbm.at[b0 + bb - 1],
              osems[1 - par]).wait()
        fire_gather(bb + 1, 1 - par, gsems[1 - par])

      for cp in gather_copies(bb, par, gsems[par]):
        cp.wait()

      def add_rows(i, c):
        r = i * 8
        for uu in range(8):
          for h in range(d // 16):
            sl = pl.ds(h * 16, 16)
            rows_v[par, r + uu, sl] = rows_v[par, r + uu, sl] + pos_v[r + uu, sl]
        return c

      lax.fori_loop(0, seq_len // 8, add_rows, 0)
      pltpu.async_copy(rows_v.at[par], out_hbm.at[b0 + bb], osems[par])

    def pair_body(p, carry):
      seq_body(p * 2, 0)
      seq_body(p * 2 + 1, 1)
      return carry

    lax.fori_loop(0, b_per_w // 2, pair_body, 0)

    for par in range(2):
      bb_last = b_per_w - 2 + par
      pltpu.make_async_copy(
          rows_v.at[par], out_hbm.at[b0 + bb_last], osems[par]).wait()

  return k(x, token_table, pos_table)


def kernel(x, token_table, pos_table):
  batch, seq_len = x.shape
  vocab, d = token_table.shape
  ident = jnp.eye(128, dtype=jnp.float32)
  table_lin = _tc_table_to_rowmajor(token_table.T, ident, vocab, d)
  table_lin = table_lin.reshape((vocab + TR_CHUNK - 1) // TR_CHUNK * TR_CHUNK,
                                d)
  cb = TR_CHUNK.bit_length() - 1
  sb = cb - 2
  x = x.astype(jnp.int32)
  xm = (((x >> cb) << cb) + ((x & ((1 << sb) - 1)) << 2)
        + ((x >> sb) & 3))
  return _sc_embed(xm, table_lin, pos_table, batch, seq_len, d)


# speedup vs baseline: 2.2812x; 1.1939x over previous
import functools

import jax
import jax.numpy as jnp
from jax import lax
from jax.experimental import pallas as pl
from jax.experimental.pallas import tpu as pltpu
from jax.experimental.pallas import tpu_sc as plsc

NUM_CORES = 2
NUM_SUBCORES = 16
NUM_WORKERS = NUM_CORES * NUM_SUBCORES

TR_CHUNK = 32768


@functools.partial(jax.jit, static_argnums=(2, 3))
def _tc_table_to_rowmajor(table_t, ident, vocab, d):
  grid = (vocab + TR_CHUNK - 1) // TR_CHUNK
  rows_per_chunk = TR_CHUNK * d // 128
  sub = TR_CHUNK // 4

  def body(ident_ref, t_ref, o_ref):
    idm = ident_ref[...]
    for j in range(sub // 128):
      ys = []
      for kq in range(4):
        xq = t_ref[:, pl.ds((kq * (sub // 128) + j) * 128, 128)]
        ys.append(lax.dot_general(idm, xq, (((1,), (1,)), ((), ())),
                                  preferred_element_type=jnp.float32))
      o_ref[pl.ds(j * 128, 128), :] = jnp.concatenate(ys, axis=1)

  return pl.pallas_call(
      body,
      grid=(grid,),
      in_specs=[pl.BlockSpec((128, 128), lambda i: (0, 0)),
                pl.BlockSpec((d, TR_CHUNK), lambda i: (0, i))],
      out_specs=pl.BlockSpec((rows_per_chunk, 128), lambda i: (i, 0)),
      out_shape=jax.ShapeDtypeStruct((grid * rows_per_chunk, 128),
                                     jnp.float32),
  )(ident, table_t)


@functools.partial(jax.jit, static_argnums=(3, 4, 5))
def _sc_embed(x, token_table, pos_table, batch, seq_len, d):
  b_per_w = batch // NUM_WORKERS
  mesh = plsc.VectorSubcoreMesh(core_axis_name="c", subcore_axis_name="s")

  @functools.partial(
      pl.kernel,
      mesh=mesh,
      compiler_params=pltpu.CompilerParams(use_tc_tiling_on_sc=False),
      out_type=jax.ShapeDtypeStruct((batch, seq_len, d), jnp.float32),
      scratch_types=[
          pltpu.VMEM((b_per_w, seq_len), jnp.int32),
          pltpu.VMEM((seq_len, d), jnp.float32),
          pltpu.VMEM((2, seq_len, d), jnp.float32),
          pltpu.SemaphoreType.DMA,
          pltpu.SemaphoreType.DMA,
          pltpu.SemaphoreType.DMA,
          pltpu.SemaphoreType.DMA,
      ],
  )
  def k(x_hbm, table_hbm, pos_hbm, out_hbm, idx_v, pos_v, rows_v,
        gsem0, gsem1, osem0, osem1):
    wid = lax.axis_index("s") * NUM_CORES + lax.axis_index("c")
    b0 = wid * b_per_w
    pltpu.sync_copy(x_hbm.at[pl.ds(b0, b_per_w), :], idx_v)
    pltpu.sync_copy(pos_hbm, pos_v)

    gsems = (gsem0, gsem1)
    osems = (osem0, osem1)

    def gather_copies(bb, par, sem):
      return (
          pltpu.make_async_copy(
              table_hbm.at[idx_v.at[bb, pl.ds(0, 128)]],
              rows_v.at[par, pl.ds(0, 128)], sem),
          pltpu.make_async_copy(
              table_hbm.at[idx_v.at[bb, pl.ds(128, seq_len - 128)]],
              rows_v.at[par, pl.ds(128, seq_len - 128)], sem),
      )

    def fire_gather(bb, par, sem):
      for cp in gather_copies(bb, par, sem):
        cp.start()

    fire_gather(0, 0, gsem0)

    def seq_body(bb, par):
      @pl.when(bb + 1 < b_per_w)
      def _():
        @pl.when(bb >= 1)
        def _():
          pltpu.make_async_copy(
              rows_v.at[1 - par], out_hbm.at[b0 + bb - 1],
              osems[1 - par]).wait()
        fire_gather(bb + 1, 1 - par, gsems[1 - par])

      for cp in gather_copies(bb, par, gsems[par]):
        cp.wait()

      def add_rows(i, c):
        r = i * 8
        for uu in range(8):
          for h in range(d // 16):
            sl = pl.ds(h * 16, 16)
            rows_v[par, r + uu, sl] = rows_v[par, r + uu, sl] + pos_v[r + uu, sl]
        return c

      lax.fori_loop(0, seq_len // 8, add_rows, 0)
      pltpu.async_copy(rows_v.at[par], out_hbm.at[b0 + bb], osems[par])

    def pair_body(p, carry):
      seq_body(p * 2, 0)
      seq_body(p * 2 + 1, 1)
      return carry

    lax.fori_loop(0, b_per_w // 2, pair_body, 0)

    for par in range(2):
      bb_last = b_per_w - 2 + par
      pltpu.make_async_copy(
          rows_v.at[par], out_hbm.at[b0 + bb_last], osems[par]).wait()

  return k(x, token_table, pos_table)


def kernel(x, token_table, pos_table):
  batch, seq_len = x.shape
  vocab, d = token_table.shape
  ident = jnp.eye(128, dtype=jnp.float32)
  table_lin = _tc_table_to_rowmajor(token_table.T, ident, vocab, d)
  table_lin = table_lin.reshape((vocab + TR_CHUNK - 1) // TR_CHUNK * TR_CHUNK,
                                d)
  cb = TR_CHUNK.bit_length() - 1
  sb = cb - 2
  x = x.astype(jnp.int32)
  xm = (((x >> cb) << cb) + ((x & ((1 << sb) - 1)) << 2)
        + ((x >> sb) & 3))
  return _sc_embed(xm, table_lin, pos_table, batch, seq_len, d)
